# Initial kernel scaffold; baseline (speedup 1.0000x reference)
import jax
import jax.numpy as jnp
from jax.experimental import pallas as pl


def kernel(x, pe, edge_index, batch, num_nodes, node_enc_w, node_enc_b, pe_enc_w, pe_enc_b, conv_w, conv_b, bn_g, bn_b, bn_m, bn_v, head_w1, head_b1, head_w2, head_b2):
    raise NotImplementedError("write your pallas kernel here")


# speedup vs baseline: 1.0036x; 1.0036x over previous
import jax
import jax.numpy as jnp
from jax.experimental import pallas as pl

_L = 3
_EPS = 1e-5
_G = 64


def _head_pk(out_ref, w1t_ref, b1_ref, res_ref):
    o = out_ref[...]
    res_ref[...] = jnp.maximum(jnp.dot(o, w1t_ref[...]) + b1_ref[...], 0.0)


def kernel(x, pe, edge_index, batch, num_nodes, node_enc_w, node_enc_b, pe_enc_w, pe_enc_b, conv_w, conv_b, bn_g, bn_b, bn_m, bn_v, head_w1, head_b1, head_w2, head_b2):
    h = (x @ node_enc_w.T + node_enc_b) + (pe @ pe_enc_w.T + pe_enc_b)
    row = jnp.minimum(edge_index[0], num_nodes - 1)
    col = edge_index[1]
    deg = jnp.zeros((h.shape[0],), dtype=h.dtype).at[row].add(1.0)
    dis = jnp.where(deg > 0, jax.lax.rsqrt(jnp.maximum(deg, 1.0)), 0.0)
    norm = dis[row] * dis[col]
    for i in range(_L):
        msg = h[col] * norm[:, None]
        agg = jnp.zeros_like(h).at[row].add(msg) + h
        h_new = agg @ conv_w[i].T + conv_b[i]
        h_new = (h_new - bn_m[i]) / jnp.sqrt(bn_v[i] + _EPS) * bn_g[i] + bn_b[i]
        h = h + jax.nn.relu(h_new)
    out = jax.ops.segment_sum(h, batch, num_segments=_G)
    count = jax.ops.segment_sum(jnp.ones((h.shape[0], 1), h.dtype), batch, num_segments=_G)
    out = out / jnp.clip(count, 1.0, None)
    z = pl.pallas_call(
        _head_pk,
        out_shape=jax.ShapeDtypeStruct((_G, 48), jnp.float32),
    )(out, head_w1.T, head_b1[None, :])
    res = z @ head_w2.T + head_b2
    return res.squeeze(-1)


# speedup vs baseline: 14.8014x; 14.7477x over previous
import functools

import jax
import jax.numpy as jnp
from jax import lax
from jax.experimental import pallas as pl
from jax.experimental.pallas import tpu as pltpu
from jax.experimental.pallas import tpu_sc as plsc

N = 100000
E = 1600000
C = 48
GW = 16
NG = 3
NSEG = 64
EPS = 1e-5

NC = 2
NS = 16
NPAD = 100352
TSLICE = NPAD // NS
EB = 1000
NBLK = E // EB
BN = 1000
NBN = N // BN

_mesh = plsc.VectorSubcoreMesh(
    core_axis_name="c", subcore_axis_name="s", num_cores=NC, num_subcores=NS)


def _zero_acc(zeros_hbm, acc, t):
    pltpu.sync_copy(zeros_hbm.at[pl.ds(t * TSLICE, TSLICE)],
                    acc.at[pl.ds(t * TSLICE, TSLICE)])


@functools.partial(
    pl.kernel,
    out_type=jax.ShapeDtypeStruct((NC, NPAD, GW), jnp.float32),
    mesh=_mesh,
    compiler_params=pltpu.CompilerParams(use_tc_tiling_on_sc=False),
    scratch_types=[
        pltpu.VMEM_SHARED((NPAD, GW), jnp.float32),
        pltpu.VMEM((EB,), jnp.int32),
        pltpu.VMEM((EB, GW), jnp.float32),
    ],
)
def _deg_kernel(row_hbm, ones_hbm, zeros_hbm, out_hbm, acc, idx_v, ones_v):
    c = lax.axis_index("c")
    s = lax.axis_index("s")
    _zero_acc(zeros_hbm, acc, s)
    pltpu.sync_copy(ones_hbm, ones_v)
    plsc.subcore_barrier()

    wid = c * NS + s

    def body(j, _):
        k = wid + j * (NC * NS)
        pltpu.sync_copy(row_hbm.at[pl.ds(k * EB, EB)], idx_v)
        pltpu.sync_copy(ones_v, acc.at[idx_v], add=True)
        return _

    lax.fori_loop(0, NBLK // (NC * NS), body, None)
    plsc.subcore_barrier()
    pltpu.sync_copy(acc.at[pl.ds(s * TSLICE, TSLICE)],
                    out_hbm.at[c].at[pl.ds(s * TSLICE, TSLICE)])


@functools.partial(
    pl.kernel,
    out_type=jax.ShapeDtypeStruct((4, NPAD, GW), jnp.float32),
    mesh=_mesh,
    compiler_params=pltpu.CompilerParams(use_tc_tiling_on_sc=False),
    scratch_types=[
        pltpu.VMEM_SHARED((NPAD, GW), jnp.float32),
        pltpu.VMEM((EB,), jnp.int32),
        pltpu.VMEM((EB,), jnp.int32),
        pltpu.VMEM((EB, GW), jnp.float32),
        pltpu.SemaphoreType.DMA,
    ],
)
def _gs_kernel(u_hbm, col_hbm, row_hbm, zeros_hbm, s_hbm,
               acc, idxc_v, idxr_v, rows_v, sem):
    c = lax.axis_index("c")
    s = lax.axis_index("s")

    def edge_block(k, g):
        pltpu.sync_copy(col_hbm.at[pl.ds(k * EB, EB)], idxc_v)
        pltpu.async_copy(u_hbm.at[g].at[idxc_v], rows_v, sem).wait()
        pltpu.sync_copy(row_hbm.at[pl.ds(k * EB, EB)], idxr_v)
        pltpu.sync_copy(rows_v, acc.at[idxr_v], add=True)

    _zero_acc(zeros_hbm, acc, s)
    plsc.subcore_barrier()

    def body_a(j, _):
        edge_block(s + j * NS, c)
        return _

    lax.fori_loop(0, NBLK // NS, body_a, None)
    plsc.subcore_barrier()
    pltpu.sync_copy(acc.at[pl.ds(s * TSLICE, TSLICE)],
                    s_hbm.at[c].at[pl.ds(s * TSLICE, TSLICE)])

    _zero_acc(zeros_hbm, acc, s)
    plsc.subcore_barrier()
    half = NBLK // 2

    def body_b(j, _):
        edge_block(c * half + s + j * NS, 2)
        return _

    lax.fori_loop(0, half // NS, body_b, None)
    plsc.subcore_barrier()
    pltpu.sync_copy(acc.at[pl.ds(s * TSLICE, TSLICE)],
                    s_hbm.at[2 + c].at[pl.ds(s * TSLICE, TSLICE)])


def _enc_body(xpe_ref, dpart_ref, encwt_ref, encb_ref,
              h_ref, u_ref, dis_ref):
    deg = dpart_ref[0] + dpart_ref[1]
    dis16 = jnp.where(deg > 0, lax.rsqrt(jnp.maximum(deg, 1.0)), 0.0)
    dis48 = jnp.concatenate([dis16, dis16, dis16], axis=1)
    h0 = jnp.dot(xpe_ref[...], encwt_ref[...],
                 preferred_element_type=jnp.float32) + encb_ref[...]
    h_ref[...] = h0
    dis_ref[...] = dis16
    u48 = dis48 * h0
    u_ref[0] = u48[:, 0:GW]
    u_ref[1] = u48[:, GW:2 * GW]
    u_ref[2] = u48[:, 2 * GW:3 * GW]


def _layer_body(s_ref, h_ref, dis_ref, wt_ref, b_ref, hn_ref, un_ref):
    dis16 = dis_ref[...]
    dis48 = jnp.concatenate([dis16, dis16, dis16], axis=1)
    s48 = jnp.concatenate([s_ref[0], s_ref[1], s_ref[2] + s_ref[3]], axis=1)
    agg = dis48 * s48 + h_ref[...]
    z = jnp.dot(agg, wt_ref[...], preferred_element_type=jnp.float32) + b_ref[...]
    hn = h_ref[...] + jnp.maximum(z, 0.0)
    hn_ref[...] = hn
    u48 = dis48 * hn
    un_ref[0] = u48[:, 0:GW]
    un_ref[1] = u48[:, GW:2 * GW]
    un_ref[2] = u48[:, 2 * GW:3 * GW]


def _layer3_pool_body(s_ref, h_ref, dis_ref, wt_ref, b_ref, batch_ref,
                      hw1t_ref, hb1_ref, hw2_ref, hb2_ref,
                      res_ref, pool_acc, cnt_acc):
    i = pl.program_id(0)
    dis16 = dis_ref[...]
    dis48 = jnp.concatenate([dis16, dis16, dis16], axis=1)
    s48 = jnp.concatenate([s_ref[0], s_ref[1], s_ref[2] + s_ref[3]], axis=1)
    agg = dis48 * s48 + h_ref[...]
    z = jnp.dot(agg, wt_ref[...], preferred_element_type=jnp.float32) + b_ref[...]
    hn = h_ref[...] + jnp.maximum(z, 0.0)

    @pl.when(i == 0)
    def _():
        pool_acc[...] = jnp.zeros_like(pool_acc)
        cnt_acc[...] = jnp.zeros_like(cnt_acc)

    b = batch_ref[...].reshape(1, BN)
    seg = lax.broadcasted_iota(jnp.int32, (NSEG, BN), 0)
    oh = jnp.where(seg == b, 1.0, 0.0)
    pool_acc[...] += jnp.dot(oh, hn, preferred_element_type=jnp.float32)
    cnt_acc[...] += jnp.dot(oh, jnp.ones((BN, C), jnp.float32),
                            preferred_element_type=jnp.float32)

    @pl.when(i == NBN - 1)
    def _():
        mean = pool_acc[...] / jnp.maximum(cnt_acc[...], 1.0)
        z2 = jnp.maximum(
            jnp.dot(mean, hw1t_ref[...], preferred_element_type=jnp.float32)
            + hb1_ref[...], 0.0)
        r = jnp.sum(z2 * hw2_ref[...], axis=1, keepdims=True) + hb2_ref[...]
        res_ref[...] = r


def _enc_call(xpe, dpart, encwt, encb):
    return pl.pallas_call(
        _enc_body,
        grid=(NBN,),
        in_specs=[
            pl.BlockSpec((BN, 9), lambda i: (i, 0)),
            pl.BlockSpec((NC, BN, GW), lambda i: (0, i, 0)),
            pl.BlockSpec((9, C), lambda i: (0, 0)),
            pl.BlockSpec((1, C), lambda i: (0, 0)),
        ],
        out_specs=[
            pl.BlockSpec((BN, C), lambda i: (i, 0)),
            pl.BlockSpec((NG, BN, GW), lambda i: (0, i, 0)),
            pl.BlockSpec((BN, GW), lambda i: (i, 0)),
        ],
        out_shape=[
            jax.ShapeDtypeStruct((N, C), jnp.float32),
            jax.ShapeDtypeStruct((NG, N, GW), jnp.float32),
            jax.ShapeDtypeStruct((N, GW), jnp.float32),
        ],
    )(xpe, dpart, encwt, encb)


def _layer_call(svec, h, dis16, wt, b):
    return pl.pallas_call(
        _layer_body,
        grid=(NBN,),
        in_specs=[
            pl.BlockSpec((4, BN, GW), lambda i: (0, i, 0)),
            pl.BlockSpec((BN, C), lambda i: (i, 0)),
            pl.BlockSpec((BN, GW), lambda i: (i, 0)),
            pl.BlockSpec((C, C), lambda i: (0, 0)),
            pl.BlockSpec((1, C), lambda i: (0, 0)),
        ],
        out_specs=[
            pl.BlockSpec((BN, C), lambda i: (i, 0)),
            pl.BlockSpec((NG, BN, GW), lambda i: (0, i, 0)),
        ],
        out_shape=[
            jax.ShapeDtypeStruct((N, C), jnp.float32),
            jax.ShapeDtypeStruct((NG, N, GW), jnp.float32),
        ],
    )(svec, h, dis16, wt, b)


def _layer3_pool_call(svec, h, dis16, wt, b, batch3d, hw1t, hb1, hw2, hb2):
    return pl.pallas_call(
        _layer3_pool_body,
        grid=(NBN,),
        in_specs=[
            pl.BlockSpec((4, BN, GW), lambda i: (0, i, 0)),
            pl.BlockSpec((BN, C), lambda i: (i, 0)),
            pl.BlockSpec((BN, GW), lambda i: (i, 0)),
            pl.BlockSpec((C, C), lambda i: (0, 0)),
            pl.BlockSpec((1, C), lambda i: (0, 0)),
            pl.BlockSpec((1, 1, BN), lambda i: (i, 0, 0)),
            pl.BlockSpec((C, C), lambda i: (0, 0)),
            pl.BlockSpec((1, C), lambda i: (0, 0)),
            pl.BlockSpec((1, C), lambda i: (0, 0)),
            pl.BlockSpec((1, 1), lambda i: (0, 0)),
        ],
        out_specs=pl.BlockSpec((NSEG, 1), lambda i: (0, 0)),
        out_shape=jax.ShapeDtypeStruct((NSEG, 1), jnp.float32),
        scratch_shapes=[
            pltpu.VMEM((NSEG, C), jnp.float32),
            pltpu.VMEM((NSEG, C), jnp.float32),
        ],
    )(svec, h, dis16, wt, b, batch3d, hw1t, hb1, hw2, hb2)


def kernel(x, pe, edge_index, batch, num_nodes, node_enc_w, node_enc_b,
           pe_enc_w, pe_enc_b, conv_w, conv_b, bn_g, bn_b, bn_m, bn_v,
           head_w1, head_b1, head_w2, head_b2):
    row = jnp.minimum(edge_index[0], num_nodes - 1)
    col = edge_index[1]

    ones2d = jnp.ones((EB, GW), jnp.float32)
    zeros2d = jnp.zeros((NPAD, GW), jnp.float32)

    inv = bn_g / jnp.sqrt(bn_v + EPS)
    wts = [(conv_w[i] * inv[i][:, None]).T for i in range(NG)]
    bs = [((conv_b[i] - bn_m[i]) * inv[i] + bn_b[i])[None, :] for i in range(NG)]

    xpe = jnp.concatenate([x, pe], axis=1)
    encwt = jnp.concatenate([node_enc_w, pe_enc_w], axis=1).T
    encb = (node_enc_b + pe_enc_b)[None, :]

    dpart = _deg_kernel(row, ones2d, zeros2d)
    h, u, dis16 = _enc_call(xpe, dpart, encwt, encb)

    for i in range(NG - 1):
        svec = _gs_kernel(u, col, row, zeros2d)
        h, u = _layer_call(svec, h, dis16, wts[i], bs[i])

    svec = _gs_kernel(u, col, row, zeros2d)
    res = _layer3_pool_call(
        svec, h, dis16, wts[2], bs[2],
        batch.reshape(NBN, 1, BN), head_w1.T, head_b1[None, :],
        head_w2, head_b2.reshape(1, 1))
    return res.squeeze(-1)
